# baseline (speedup 1.0000x reference)
import jax
import jax.numpy as jnp
from jax.experimental import pallas as pl
from jax.experimental.pallas import tpu as pltpu

_T_CHUNK = 128
_UNROLL = 16


def _cayley_kernel(s_ref, dt_ref, xa, rr):
    d = s_ref.shape[0]
    rows = jax.lax.broadcasted_iota(jnp.int32, (d, d), 0)
    cols = jax.lax.broadcasted_iota(jnp.int32, (d, d), 1)
    eye = jnp.where(rows == cols, 1.0, 0.0).astype(jnp.float32)
    hi = jax.lax.Precision.HIGHEST
    xa[...] = eye + s_ref[...]
    for it in range(6):
        prec = None if it < 4 else hi
        rr[...] = jnp.dot(eye - s_ref[...], xa[...],
                          preferred_element_type=jnp.float32, precision=prec)
        xa[...] = jnp.dot(xa[...], 2.0 * eye - rr[...],
                          preferred_element_type=jnp.float32, precision=prec)
    dt_ref[...] = jnp.dot(xa[...], eye + s_ref[...],
                          preferred_element_type=jnp.float32, precision=hi)


def _scan_kernel(xs_ref, w1_ref, at_ref, gw_ref, gb_ref, ww_ref,
                 wb_ref, gam_ref, out_ref, g_s, u_s, h_s):
    j = pl.program_id(0)
    t_chunk, bh, d = xs_ref.shape
    x = xs_ref[...].reshape(t_chunk * bh, d).astype(jnp.bfloat16)
    gd = jax.nn.sigmoid(gam_ref[...])
    g_s[...] = jax.nn.sigmoid(
        jnp.dot(x, gw_ref[...], preferred_element_type=jnp.float32)
        + gb_ref[...])
    u_s[...] = (jnp.dot(x, ww_ref[...], preferred_element_type=jnp.float32)
                + wb_ref[...])

    @pl.when(j == 0)
    def _():
        h_s[...] = jnp.zeros_like(h_s)

    u = _UNROLL

    def body(k, carry):
        rows = u * bh
        g_blk = g_s[pl.ds(k * rows, rows), :]
        u_blk = u_s[pl.ds(k * rows, rows), :]
        h = h_s[...]
        for s in range(u):
            g_t = g_blk[s * bh:(s + 1) * bh, :]
            u_t = u_blk[s * bh:(s + 1) * bh, :]
            yt = jnp.dot(w1_ref[...], h.T, preferred_element_type=jnp.float32)
            hp = yt.T
            hd = hp[:, :d]
            p = hp[:, d:d + at_ref.shape[0]]
            low = jnp.dot(p, at_ref[...], preferred_element_type=jnp.float32)
            h = gd * hd + g_t * low + u_t
            out_ref[k * u + s] = h
        h_s[...] = h
        return carry

    jax.lax.fori_loop(0, t_chunk // u, body, 0)


def kernel(x_seq, S_params, gamma, A, B, gate_w, gate_b, win_w, win_b):
    bsz, seq, d = x_seq.shape
    r = A.shape[1]
    f32 = jnp.float32

    skew = (S_params - S_params.T) * 0.5
    dt = pl.pallas_call(
        _cayley_kernel,
        out_shape=jax.ShapeDtypeStruct((d, d), f32),
        scratch_shapes=[pltpu.VMEM((d, d), f32), pltpu.VMEM((d, d), f32)],
        compiler_params=pltpu.CompilerParams(vmem_limit_bytes=50 * 1024 * 1024),
    )(skew)

    w1 = jnp.concatenate([dt, B.T, jnp.zeros((d, 128 - r), f32)], axis=1)

    t = _T_CHUNK
    xs = jnp.transpose(x_seq, (1, 0, 2))
    grid = (seq // t,)

    full = lambda shape: pl.BlockSpec(shape, lambda j: (0,) * len(shape))
    out_t = pl.pallas_call(
        _scan_kernel,
        out_shape=jax.ShapeDtypeStruct((seq, bsz, d), f32),
        grid=grid,
        in_specs=[
            pl.BlockSpec((t, bsz, d), lambda j: (j, 0, 0)),
            full((d + 128, d)),
            full((r, d)),
            full((d, d)),
            full((1, d)),
            full((d, d)),
            full((1, d)),
            full((1, 1)),
        ],
        out_specs=pl.BlockSpec((t, bsz, d), lambda j: (j, 0, 0)),
        scratch_shapes=[
            pltpu.VMEM((t * bsz, d), f32),
            pltpu.VMEM((t * bsz, d), f32),
            pltpu.VMEM((bsz, d), f32),
        ],
        compiler_params=pltpu.CompilerParams(
            dimension_semantics=("arbitrary",),
            vmem_limit_bytes=50 * 1024 * 1024,
        ),
        name="holo_net_vault_scan",
    )(xs, w1.T, A.T, gate_w.T.astype(jnp.bfloat16), gate_b.reshape(1, d),
      win_w.T.astype(jnp.bfloat16), win_b.reshape(1, d), gamma.reshape(1, 1))

    return jnp.transpose(out_t, (1, 0, 2))


# speedup vs baseline: 2.6211x; 2.6211x over previous
import jax
import jax.numpy as jnp
from jax.experimental import pallas as pl
from jax.experimental.pallas import tpu as pltpu

_T_CHUNK = 128
_UNROLL = 16


def _cayley_kernel(s_ref, dt_ref, xa, rr):
    d = s_ref.shape[0]
    rows = jax.lax.broadcasted_iota(jnp.int32, (d, d), 0)
    cols = jax.lax.broadcasted_iota(jnp.int32, (d, d), 1)
    eye = jnp.where(rows == cols, 1.0, 0.0).astype(jnp.float32)
    hi = jax.lax.Precision.HIGHEST
    xa[...] = eye + s_ref[...]
    for it in range(6):
        prec = None if it < 4 else hi
        rr[...] = jnp.dot(eye - s_ref[...], xa[...],
                          preferred_element_type=jnp.float32, precision=prec)
        xa[...] = jnp.dot(xa[...], 2.0 * eye - rr[...],
                          preferred_element_type=jnp.float32, precision=prec)
    dt_ref[...] = jnp.dot(xa[...], eye + s_ref[...],
                          preferred_element_type=jnp.float32, precision=hi)


def _scan_kernel(xs_ref, w1_ref, at_ref, gw_ref, gb_ref, ww_ref,
                 wb_ref, gam_ref, out_ref, g_s, u_s, h_s):
    j = pl.program_id(0)
    t_chunk, bh, d = xs_ref.shape
    x = xs_ref[...].reshape(t_chunk * bh, d).astype(jnp.bfloat16)
    gd = jax.nn.sigmoid(gam_ref[...])
    g_s[...] = jax.nn.sigmoid(
        jnp.dot(x, gw_ref[...], preferred_element_type=jnp.float32)
        + gb_ref[...])
    u_s[...] = (jnp.dot(x, ww_ref[...], preferred_element_type=jnp.float32)
                + wb_ref[...])

    @pl.when(j == 0)
    def _():
        h_s[...] = jnp.zeros_like(h_s)

    u = _UNROLL

    def body(k, carry):
        rows = u * bh
        g_blk = g_s[pl.ds(k * rows, rows), :]
        u_blk = u_s[pl.ds(k * rows, rows), :]
        h = h_s[...]
        for s in range(u):
            g_t = g_blk[s * bh:(s + 1) * bh, :]
            u_t = u_blk[s * bh:(s + 1) * bh, :]
            hb = h.astype(jnp.bfloat16)
            hp = jnp.dot(hb, w1_ref[...], preferred_element_type=jnp.float32)
            hd = hp[:, :d]
            p = hp[:, d:d + at_ref.shape[0]]
            low = jnp.dot(p.astype(jnp.bfloat16), at_ref[...],
                          preferred_element_type=jnp.float32)
            h = gd * hd + g_t * low + u_t
            out_ref[k * u + s] = h
        h_s[...] = h
        return carry

    jax.lax.fori_loop(0, t_chunk // u, body, 0)


def kernel(x_seq, S_params, gamma, A, B, gate_w, gate_b, win_w, win_b):
    bsz, seq, d = x_seq.shape
    r = A.shape[1]
    f32 = jnp.float32

    skew = (S_params - S_params.T) * 0.5
    dt = pl.pallas_call(
        _cayley_kernel,
        out_shape=jax.ShapeDtypeStruct((d, d), f32),
        scratch_shapes=[pltpu.VMEM((d, d), f32), pltpu.VMEM((d, d), f32)],
        compiler_params=pltpu.CompilerParams(vmem_limit_bytes=50 * 1024 * 1024),
    )(skew)

    w1 = jnp.concatenate([dt, B.T, jnp.zeros((d, 128 - r), f32)], axis=1)

    t = _T_CHUNK
    xs = jnp.transpose(x_seq, (1, 0, 2))
    grid = (seq // t,)

    full = lambda shape: pl.BlockSpec(shape, lambda j: (0,) * len(shape))
    out_t = pl.pallas_call(
        _scan_kernel,
        out_shape=jax.ShapeDtypeStruct((seq, bsz, d), f32),
        grid=grid,
        in_specs=[
            pl.BlockSpec((t, bsz, d), lambda j: (j, 0, 0)),
            full((d, d + 128)),
            full((r, d)),
            full((d, d)),
            full((1, d)),
            full((d, d)),
            full((1, d)),
            full((1, 1)),
        ],
        out_specs=pl.BlockSpec((t, bsz, d), lambda j: (j, 0, 0)),
        scratch_shapes=[
            pltpu.VMEM((t * bsz, d), f32),
            pltpu.VMEM((t * bsz, d), f32),
            pltpu.VMEM((bsz, d), f32),
        ],
        compiler_params=pltpu.CompilerParams(
            dimension_semantics=("arbitrary",),
            vmem_limit_bytes=50 * 1024 * 1024,
        ),
        name="holo_net_vault_scan",
    )(xs, w1.astype(jnp.bfloat16), A.T.astype(jnp.bfloat16), gate_w.T.astype(jnp.bfloat16), gate_b.reshape(1, d),
      win_w.T.astype(jnp.bfloat16), win_b.reshape(1, d), gamma.reshape(1, 1))

    return jnp.transpose(out_t, (1, 0, 2))


# speedup vs baseline: 2.6972x; 1.0290x over previous
import jax
import jax.numpy as jnp
from jax.experimental import pallas as pl
from jax.experimental.pallas import tpu as pltpu

_T_CHUNK = 128
_UNROLL = 16


def _cayley_kernel(s_ref, dt_ref, xa, rr):
    d = s_ref.shape[0]
    rows = jax.lax.broadcasted_iota(jnp.int32, (d, d), 0)
    cols = jax.lax.broadcasted_iota(jnp.int32, (d, d), 1)
    eye = jnp.where(rows == cols, 1.0, 0.0).astype(jnp.float32)
    hi = jax.lax.Precision.HIGHEST
    xa[...] = eye + s_ref[...]
    for it in range(6):
        prec = None if it < 4 else hi
        rr[...] = jnp.dot(eye - s_ref[...], xa[...],
                          preferred_element_type=jnp.float32, precision=prec)
        xa[...] = jnp.dot(xa[...], 2.0 * eye - rr[...],
                          preferred_element_type=jnp.float32, precision=prec)
    dt_ref[...] = jnp.dot(xa[...], eye + s_ref[...],
                          preferred_element_type=jnp.float32, precision=hi)


def _scan_kernel(xs_ref, w1_ref, at_ref, gw_ref, gb_ref, ww_ref,
                 wb_ref, gam_ref, out_ref, g_s, u_s, h_s):
    j = pl.program_id(0)
    t_chunk, bh, d = xs_ref.shape
    x = xs_ref[...].reshape(t_chunk * bh, d).astype(jnp.bfloat16)
    gd = jax.nn.sigmoid(gam_ref[...])
    g_s[...] = jax.nn.sigmoid(
        jnp.dot(x, gw_ref[...], preferred_element_type=jnp.float32)
        + gb_ref[...])
    u_s[...] = (jnp.dot(x, ww_ref[...], preferred_element_type=jnp.float32)
                + wb_ref[...])

    @pl.when(j == 0)
    def _():
        h_s[...] = jnp.zeros_like(h_s)

    u = _UNROLL

    def body(k, carry):
        rows = u * bh
        g_blk = g_s[pl.ds(k * rows, rows), :]
        u_blk = u_s[pl.ds(k * rows, rows), :]
        h = h_s[...]
        for s in range(u):
            g_t = g_blk[s * bh:(s + 1) * bh, :]
            u_t = u_blk[s * bh:(s + 1) * bh, :]
            hb = h.astype(jnp.bfloat16)
            hp = jnp.dot(hb, w1_ref[...], preferred_element_type=jnp.float32)
            hd = hp[:, :d]
            p = hp[:, d:d + at_ref.shape[0]]
            low = p[:, 0:1] * at_ref[0:1, :]
            for jj in range(1, at_ref.shape[0]):
                low = low + p[:, jj:jj + 1] * at_ref[jj:jj + 1, :]
            h = gd * hd + g_t * low + u_t
            out_ref[k * u + s] = h
        h_s[...] = h
        return carry

    jax.lax.fori_loop(0, t_chunk // u, body, 0)


def kernel(x_seq, S_params, gamma, A, B, gate_w, gate_b, win_w, win_b):
    bsz, seq, d = x_seq.shape
    r = A.shape[1]
    f32 = jnp.float32

    skew = (S_params - S_params.T) * 0.5
    dt = pl.pallas_call(
        _cayley_kernel,
        out_shape=jax.ShapeDtypeStruct((d, d), f32),
        scratch_shapes=[pltpu.VMEM((d, d), f32), pltpu.VMEM((d, d), f32)],
        compiler_params=pltpu.CompilerParams(vmem_limit_bytes=50 * 1024 * 1024),
    )(skew)

    w1 = jnp.concatenate([dt, B.T, jnp.zeros((d, 128 - r), f32)], axis=1)

    t = _T_CHUNK
    xs = jnp.transpose(x_seq, (1, 0, 2))
    grid = (seq // t,)

    full = lambda shape: pl.BlockSpec(shape, lambda j: (0,) * len(shape))
    out_t = pl.pallas_call(
        _scan_kernel,
        out_shape=jax.ShapeDtypeStruct((seq, bsz, d), f32),
        grid=grid,
        in_specs=[
            pl.BlockSpec((t, bsz, d), lambda j: (j, 0, 0)),
            full((d, d + 128)),
            full((r, d)),
            full((d, d)),
            full((1, d)),
            full((d, d)),
            full((1, d)),
            full((1, 1)),
        ],
        out_specs=pl.BlockSpec((t, bsz, d), lambda j: (j, 0, 0)),
        scratch_shapes=[
            pltpu.VMEM((t * bsz, d), f32),
            pltpu.VMEM((t * bsz, d), f32),
            pltpu.VMEM((bsz, d), f32),
        ],
        compiler_params=pltpu.CompilerParams(
            dimension_semantics=("arbitrary",),
            vmem_limit_bytes=50 * 1024 * 1024,
        ),
        name="holo_net_vault_scan",
    )(xs, w1.astype(jnp.bfloat16), A.T, gate_w.T.astype(jnp.bfloat16), gate_b.reshape(1, d),
      win_w.T.astype(jnp.bfloat16), win_b.reshape(1, d), gamma.reshape(1, 1))

    return jnp.transpose(out_t, (1, 0, 2))
